# Initial kernel scaffold; baseline (speedup 1.0000x reference)
import jax
import jax.numpy as jnp
from jax.experimental import pallas as pl


def kernel(x, emb_table):
    raise NotImplementedError("write your pallas kernel here")


# speedup vs baseline: 1.2912x; 1.2912x over previous
import jax
import jax.numpy as jnp
from jax.experimental import pallas as pl

_S_BLK = 256


def _add_kernel(x_ref, e_ref, o_ref):
    o_ref[...] = x_ref[...] + e_ref[...][None, :, :]


def kernel(x, emb_table):
    batch, seq_len, hidden = x.shape
    s_blk = _S_BLK if seq_len % _S_BLK == 0 else seq_len
    grid = (seq_len // s_blk, batch)
    return pl.pallas_call(
        _add_kernel,
        grid=grid,
        in_specs=[
            pl.BlockSpec((1, s_blk, hidden), lambda s, b: (b, s, 0)),
            pl.BlockSpec((s_blk, hidden), lambda s, b: (s, 0)),
        ],
        out_specs=pl.BlockSpec((1, s_blk, hidden), lambda s, b: (b, s, 0)),
        out_shape=jax.ShapeDtypeStruct((batch, seq_len, hidden), x.dtype),
    )(x, emb_table[:seq_len])


# speedup vs baseline: 1.6814x; 1.3022x over previous
import jax
import jax.numpy as jnp
from jax.experimental import pallas as pl

_S_BLK = 512


def _add_kernel(x_ref, e_ref, o_ref):
    o_ref[...] = x_ref[...] + e_ref[...][None, :, :]


def kernel(x, emb_table):
    batch, seq_len, hidden = x.shape
    s_blk = _S_BLK if seq_len % _S_BLK == 0 else seq_len
    grid = (seq_len // s_blk, batch)
    return pl.pallas_call(
        _add_kernel,
        grid=grid,
        in_specs=[
            pl.BlockSpec((1, s_blk, hidden), lambda s, b: (b, s, 0)),
            pl.BlockSpec((s_blk, hidden), lambda s, b: (s, 0)),
        ],
        out_specs=pl.BlockSpec((1, s_blk, hidden), lambda s, b: (b, s, 0)),
        out_shape=jax.ShapeDtypeStruct((batch, seq_len, hidden), x.dtype),
    )(x, emb_table[:seq_len])


# speedup vs baseline: 1.8463x; 1.0981x over previous
import jax
import jax.numpy as jnp
from jax.experimental import pallas as pl

_S_BLK = 1024


def _add_kernel(x_ref, e_ref, o_ref):
    o_ref[...] = x_ref[...] + e_ref[...][None, :, :]


def kernel(x, emb_table):
    batch, seq_len, hidden = x.shape
    s_blk = _S_BLK if seq_len % _S_BLK == 0 else seq_len
    grid = (seq_len // s_blk, batch)
    return pl.pallas_call(
        _add_kernel,
        grid=grid,
        in_specs=[
            pl.BlockSpec((1, s_blk, hidden), lambda s, b: (b, s, 0)),
            pl.BlockSpec((s_blk, hidden), lambda s, b: (s, 0)),
        ],
        out_specs=pl.BlockSpec((1, s_blk, hidden), lambda s, b: (b, s, 0)),
        out_shape=jax.ShapeDtypeStruct((batch, seq_len, hidden), x.dtype),
    )(x, emb_table[:seq_len])


# speedup vs baseline: 1.9644x; 1.0640x over previous
import jax
import jax.numpy as jnp
from jax.experimental import pallas as pl

_S_BLK = 2048


def _add_kernel(x_ref, e_ref, o_ref):
    o_ref[...] = x_ref[...] + e_ref[...][None, :, :]


def kernel(x, emb_table):
    batch, seq_len, hidden = x.shape
    s_blk = _S_BLK if seq_len % _S_BLK == 0 else seq_len
    grid = (seq_len // s_blk, batch)
    return pl.pallas_call(
        _add_kernel,
        grid=grid,
        in_specs=[
            pl.BlockSpec((1, s_blk, hidden), lambda s, b: (b, s, 0)),
            pl.BlockSpec((s_blk, hidden), lambda s, b: (s, 0)),
        ],
        out_specs=pl.BlockSpec((1, s_blk, hidden), lambda s, b: (b, s, 0)),
        out_shape=jax.ShapeDtypeStruct((batch, seq_len, hidden), x.dtype),
    )(x, emb_table[:seq_len])


# speedup vs baseline: 1.9705x; 1.0031x over previous
import jax
import jax.numpy as jnp
from jax.experimental import pallas as pl
from jax.experimental.pallas import tpu as pltpu

_S_BLK = 2048


def _add_kernel(x_ref, e_ref, o_ref):
    o_ref[...] = x_ref[...] + e_ref[...][None, :, :]


def kernel(x, emb_table):
    batch, seq_len, hidden = x.shape
    s_blk = _S_BLK if seq_len % _S_BLK == 0 else seq_len
    grid = (seq_len // s_blk, batch)
    return pl.pallas_call(
        _add_kernel,
        grid=grid,
        in_specs=[
            pl.BlockSpec((1, s_blk, hidden), lambda s, b: (b, s, 0)),
            pl.BlockSpec((s_blk, hidden), lambda s, b: (s, 0)),
        ],
        out_specs=pl.BlockSpec((1, s_blk, hidden), lambda s, b: (b, s, 0)),
        out_shape=jax.ShapeDtypeStruct((batch, seq_len, hidden), x.dtype),
        compiler_params=pltpu.CompilerParams(
            dimension_semantics=("parallel", "parallel"),
        ),
    )(x, emb_table[:seq_len])
